# baseline (speedup 1.0000x reference)
import jax
import jax.numpy as jnp
from jax.experimental import pallas as pl
from jax.experimental.pallas import tpu as pltpu

NUM_SEL = 5
BLOCK_N = 1024


def _fused_body(v_ref, w_ref, hbm_ref, o_ref, vals_ref, idxs_ref, sem):
    i = pl.program_id(0)
    j = pl.program_id(1)
    nb = pl.num_programs(1)

    @pl.when(j == 0)
    def _init():
        vals_ref[...] = jnp.full((1, 128), -jnp.inf, dtype=jnp.float32)
        idxs_ref[...] = jnp.zeros((1, 128), dtype=jnp.int32)

    v = v_ref[0]
    w = w_ref[...]
    s = jax.lax.dot_general(w, v, (((1,), (1,)), ((), ())),
                            precision=jax.lax.Precision.HIGHEST,
                            preferred_element_type=jnp.float32)

    lane = jax.lax.broadcasted_iota(jnp.int32, (1, 128), 1)
    iota = jax.lax.broadcasted_iota(jnp.int32, (1, BLOCK_N), 1)

    thresh = jnp.max(jnp.where(lane == NUM_SEL - 1, vals_ref[...], -jnp.inf))
    bm = jnp.max(s)

    @pl.when(bm > thresh)
    def _merge():
        sv = s
        nv = vals_ref[...]
        ni = idxs_ref[...]
        gbase = j * BLOCK_N
        for k in range(NUM_SEL):
            m = jnp.max(sv)
            am = jnp.min(jnp.where(sv == m, iota, BLOCK_N))
            nv = jnp.where(lane == NUM_SEL + k, m, nv)
            ni = jnp.where(lane == NUM_SEL + k, gbase + am, ni)
            sv = jnp.where(iota == am, -jnp.inf, sv)
        rv = jnp.full((1, 128), -jnp.inf, dtype=jnp.float32)
        ri = jnp.zeros((1, 128), dtype=jnp.int32)
        tv = nv
        for k in range(NUM_SEL):
            m = jnp.max(tv)
            pos = jnp.min(jnp.where(tv == m, lane, 128))
            pi = jnp.max(jnp.where(lane == pos, ni, jnp.int32(-2147483648)))
            rv = jnp.where(lane == k, m, rv)
            ri = jnp.where(lane == k, pi, ri)
            tv = jnp.where(lane == pos, -jnp.inf, tv)
        vals_ref[...] = rv
        idxs_ref[...] = ri

    @pl.when(j == nb - 1)
    def _gather():
        ii = idxs_ref[...]
        copies = []
        for k in range(NUM_SEL):
            idx_k = jnp.max(jnp.where(lane == k, ii, 0))
            c = pltpu.make_async_copy(
                hbm_ref.at[i, pl.ds(idx_k, 1), :],
                o_ref.at[i, pl.ds(k, 1), :],
                sem,
            )
            c.start()
            copies.append(c)
        for c in copies:
            c.wait()


def kernel(value, W, b):
    del b
    B, N, D = value.shape

    return pl.pallas_call(
        _fused_body,
        grid=(B, N // BLOCK_N),
        in_specs=[
            pl.BlockSpec((1, BLOCK_N, D), lambda i, j: (i, j, 0)),
            pl.BlockSpec((1, D), lambda i, j: (0, 0)),
            pl.BlockSpec(memory_space=pltpu.MemorySpace.HBM),
        ],
        out_specs=pl.BlockSpec(memory_space=pltpu.MemorySpace.HBM),
        out_shape=jax.ShapeDtypeStruct((B, NUM_SEL, D), jnp.float32),
        scratch_shapes=[
            pltpu.VMEM((1, 128), jnp.float32),
            pltpu.VMEM((1, 128), jnp.int32),
            pltpu.SemaphoreType.DMA,
        ],
    )(value, W, value)


# speedup vs baseline: 2.1303x; 2.1303x over previous
import jax
import jax.numpy as jnp
from jax.experimental import pallas as pl
from jax.experimental.pallas import tpu as pltpu

NUM_SEL = 5
BLOCK_N = 1024


def _fused_body(v_ref, w_ref, hbm_ref, o_ref, vals_ref, idxs_ref, sem):
    i = pl.program_id(0)
    j = pl.program_id(1)
    nb = pl.num_programs(1)

    @pl.when(j == 0)
    def _init():
        vals_ref[...] = jnp.full((1, 128), -jnp.inf, dtype=jnp.float32)
        idxs_ref[...] = jnp.zeros((1, 128), dtype=jnp.int32)

    v = v_ref[0]
    w = w_ref[...]
    s = jax.lax.dot_general(v, w, (((1,), (1,)), ((), ())),
                            precision=jax.lax.Precision.HIGHEST,
                            preferred_element_type=jnp.float32)
    s = s[:, 0][None, :]

    lane = jax.lax.broadcasted_iota(jnp.int32, (1, 128), 1)
    iota = jax.lax.broadcasted_iota(jnp.int32, (1, BLOCK_N), 1)

    thresh = jnp.max(jnp.where(lane == NUM_SEL - 1, vals_ref[...], -jnp.inf))
    bm = jnp.max(s)

    @pl.when(bm > thresh)
    def _merge():
        sv = s
        nv = vals_ref[...]
        ni = idxs_ref[...]
        gbase = j * BLOCK_N
        for k in range(NUM_SEL):
            m = jnp.max(sv)
            am = jnp.min(jnp.where(sv == m, iota, BLOCK_N))
            nv = jnp.where(lane == NUM_SEL + k, m, nv)
            ni = jnp.where(lane == NUM_SEL + k, gbase + am, ni)
            sv = jnp.where(iota == am, -jnp.inf, sv)
        rv = jnp.full((1, 128), -jnp.inf, dtype=jnp.float32)
        ri = jnp.zeros((1, 128), dtype=jnp.int32)
        tv = nv
        for k in range(NUM_SEL):
            m = jnp.max(tv)
            pos = jnp.min(jnp.where(tv == m, lane, 128))
            pi = jnp.max(jnp.where(lane == pos, ni, jnp.int32(-2147483648)))
            rv = jnp.where(lane == k, m, rv)
            ri = jnp.where(lane == k, pi, ri)
            tv = jnp.where(lane == pos, -jnp.inf, tv)
        vals_ref[...] = rv
        idxs_ref[...] = ri

    @pl.when(j == nb - 1)
    def _gather():
        ii = idxs_ref[...]
        copies = []
        for k in range(NUM_SEL):
            idx_k = jnp.max(jnp.where(lane == k, ii, 0))
            c = pltpu.make_async_copy(
                hbm_ref.at[i, pl.ds(idx_k, 1), :],
                o_ref.at[i, pl.ds(k, 1), :],
                sem,
            )
            c.start()
            copies.append(c)
        for c in copies:
            c.wait()


def kernel(value, W, b):
    del b
    B, N, D = value.shape

    return pl.pallas_call(
        _fused_body,
        grid=(B, N // BLOCK_N),
        in_specs=[
            pl.BlockSpec((1, BLOCK_N, D), lambda i, j: (i, j, 0)),
            pl.BlockSpec((1, D), lambda i, j: (0, 0)),
            pl.BlockSpec(memory_space=pltpu.MemorySpace.HBM),
        ],
        out_specs=pl.BlockSpec(memory_space=pltpu.MemorySpace.HBM),
        out_shape=jax.ShapeDtypeStruct((B, NUM_SEL, D), jnp.float32),
        scratch_shapes=[
            pltpu.VMEM((1, 128), jnp.float32),
            pltpu.VMEM((1, 128), jnp.int32),
            pltpu.SemaphoreType.DMA,
        ],
    )(value, W, value)
